# baseline (speedup 1.0000x reference)
import jax
import jax.numpy as jnp
from jax import lax
from jax.experimental import pallas as pl
from jax.experimental.pallas import tpu as pltpu

DIM = 128
N_ROWS = 16384
THRESH = 512.0
NO_CODE = -1
CH = 2048
NCH = N_ROWS // CH
NBUF = 3


def _nn_body(x_hbm, c_hbm, out_hbm, b0, b1, b2, c_v, out_v, sems, csem, osem):
    bufs = [b0, b1, b2]
    pltpu.make_async_copy(c_hbm.at[pl.ds(0, 8)], c_v, csem).start()
    for i in range(NBUF):
        pltpu.make_async_copy(
            x_hbm.at[pl.ds(i * CH, CH)], bufs[i], sems.at[i]
        ).start()
    pltpu.make_async_copy(c_hbm.at[pl.ds(0, 8)], c_v, csem).wait()
    cb = c_v[0:1, :].astype(jnp.bfloat16)
    ones = jnp.ones((8, DIM), jnp.bfloat16)
    for i in range(NCH):
        b = bufs[i % NBUF]
        pltpu.make_async_copy(
            x_hbm.at[pl.ds(i * CH, CH)], b, sems.at[i % NBUF]
        ).wait()
        t = b[...].astype(jnp.bfloat16) - cb
        q = t * t
        d = lax.dot_general(
            ones, q, (((1,), (1,)), ((), ())),
            preferred_element_type=jnp.float32,
        )
        out_v[pl.ds(i * CH, CH)] = jnp.where(d[0] <= THRESH, 0, NO_CODE).astype(
            jnp.int32
        )
        nxt = i + NBUF
        if nxt < NCH:
            pltpu.make_async_copy(
                x_hbm.at[pl.ds(nxt * CH, CH)], bufs[nxt % NBUF], sems.at[nxt % NBUF]
            ).start()
    pltpu.make_async_copy(out_v, out_hbm, osem).start()
    pltpu.make_async_copy(out_v, out_hbm, osem).wait()


def kernel(x, _codes):
    return pl.pallas_call(
        _nn_body,
        in_specs=[
            pl.BlockSpec(memory_space=pl.ANY),
            pl.BlockSpec(memory_space=pl.ANY),
        ],
        out_specs=pl.BlockSpec(memory_space=pl.ANY),
        out_shape=jax.ShapeDtypeStruct((N_ROWS,), jnp.int32),
        scratch_shapes=[
            pltpu.VMEM((CH, DIM), jnp.float32),
            pltpu.VMEM((CH, DIM), jnp.float32),
            pltpu.VMEM((CH, DIM), jnp.float32),
            pltpu.VMEM((8, DIM), jnp.float32),
            pltpu.VMEM((N_ROWS,), jnp.int32),
            pltpu.SemaphoreType.DMA((NBUF,)),
            pltpu.SemaphoreType.DMA,
            pltpu.SemaphoreType.DMA,
        ],
    )(x, _codes)
